# baseline (speedup 1.0000x reference)
import jax
import jax.numpy as jnp
from jax import lax
from jax.experimental import pallas as pl
from jax.experimental.pallas import tpu as pltpu

VOCAB = 1000
B = 8
K = 8


def _onehot_body(idx_ref, out_hbm, vmem, sems):
    i = pl.program_id(0)
    g = pl.num_programs(0)
    slot = lax.rem(i, K)

    def copy_for(k, step):
        return pltpu.make_async_copy(
            vmem.at[k], out_hbm.at[pl.ds(step * B, B)], sems.at[k]
        )

    for k in range(K):
        @pl.when(jnp.logical_and(slot == k, i >= K))
        def _wait(k=k):
            copy_for(k, i - K).wait()

    idx = idx_ref[...]
    iota = lax.broadcasted_iota(jnp.int32, (B, 50, VOCAB), 2)
    oh = (iota == idx[:, :, None]).astype(jnp.float32)
    for k in range(K):
        @pl.when(slot == k)
        def _compute_and_send(k=k):
            vmem[k] = oh
            copy_for(k, i).start()

    @pl.when(i == g - 1)
    def _drain():
        for j in range(K):
            step = i - (K - 1) + j

            @pl.when(step >= 0)
            def _(j=j, step=step):
                for k in range(K):
                    @pl.when(lax.rem(step, K) == k)
                    def _(k=k, step=step):
                        copy_for(k, step).wait()


def kernel(inputs):
    rows, cols = inputs.shape
    return pl.pallas_call(
        _onehot_body,
        grid=(rows // B,),
        in_specs=[pl.BlockSpec((B, cols), lambda i: (i, 0))],
        out_specs=pl.BlockSpec(memory_space=pl.ANY),
        out_shape=jax.ShapeDtypeStruct((rows, cols, VOCAB), jnp.float32),
        scratch_shapes=[
            pltpu.VMEM((K, B, cols, VOCAB), jnp.float32),
            pltpu.SemaphoreType.DMA((K,)),
        ],
        compiler_params=pltpu.CompilerParams(
            dimension_semantics=("arbitrary",),
        ),
    )(inputs)


# speedup vs baseline: 1.0171x; 1.0171x over previous
import functools
import jax
import jax.numpy as jnp
from jax import lax
from jax.experimental import pallas as pl
from jax.experimental.pallas import tpu as pltpu
from jax.experimental.pallas import tpu_sc as plsc

VOCAB = 1000
ROWS = 4096
COLS = 50
L = 16
NGROUPS = 4


def _scatter_row(idx_v, buf, c, value):
    for g in range(NGROUPS):
        col = lax.broadcasted_iota(jnp.int32, (L,), 0) + (g * L)
        idxvals = idx_v[c, pl.ds(g * L, L)]
        mask = col < COLS
        vals = jnp.full((L,), value, jnp.float32)
        plsc.store_scatter(buf, [col, idxvals], vals, mask=mask)


def _make_sc_kernel(rows_per_worker, nc, ns):
    mesh = plsc.VectorSubcoreMesh(core_axis_name="c", subcore_axis_name="s")

    @functools.partial(
        pl.kernel,
        mesh=mesh,
        out_type=jax.ShapeDtypeStruct((ROWS, COLS, VOCAB), jnp.float32),
        scratch_types=[
            pltpu.VMEM((rows_per_worker, 64), jnp.int32),
            pltpu.VMEM((COLS, VOCAB), jnp.float32),
            pltpu.VMEM((COLS, VOCAB), jnp.float32),
            pltpu.SemaphoreType.DMA,
            pltpu.SemaphoreType.DMA,
        ],
        compiler_params=pltpu.CompilerParams(needs_layout_passes=False),
    )
    def sc_onehot(idx_hbm, zeros_hbm, out_hbm, idx_v, buf0, buf1, sem0, sem1):
        wid = lax.axis_index("s") * nc + lax.axis_index("c")
        row0 = wid * rows_per_worker

        pltpu.sync_copy(idx_hbm.at[pl.ds(row0, rows_per_worker)], idx_v)
        pltpu.sync_copy(zeros_hbm, buf0)
        pltpu.sync_copy(zeros_hbm, buf1)

        def step(c, carry):
            for b, (buf, sem) in enumerate(((buf0, sem0), (buf1, sem1))):
                @pl.when(lax.rem(c, 2) == b)
                def _(buf=buf, sem=sem):
                    @pl.when(c >= 2)
                    def _():
                        pltpu.make_async_copy(
                            buf, out_hbm.at[row0 + c - 2], sem
                        ).wait()
                        _scatter_row(idx_v, buf, c - 2, 0.0)

                    _scatter_row(idx_v, buf, c, 1.0)
                    pltpu.make_async_copy(
                        buf, out_hbm.at[row0 + c], sem
                    ).start()
            return carry

        lax.fori_loop(0, rows_per_worker, step, 0)

        pltpu.make_async_copy(
            buf0, out_hbm.at[row0 + rows_per_worker - 2], sem0
        ).wait()
        pltpu.make_async_copy(
            buf1, out_hbm.at[row0 + rows_per_worker - 1], sem1
        ).wait()

    return sc_onehot


def kernel(inputs):
    info = plsc.get_sparse_core_info()
    nc, ns = info.num_cores, info.num_subcores
    nw = nc * ns
    rows_per_worker = ROWS // nw
    idx_pad = jnp.pad(inputs, ((0, 0), (0, 64 - COLS)))
    zeros = jnp.zeros((COLS, VOCAB), jnp.float32)
    sc = _make_sc_kernel(rows_per_worker, nc, ns)
    return sc(idx_pad, zeros)
